# baseline (speedup 1.0000x reference)
import functools

import jax
import jax.numpy as jnp
from jax.experimental import pallas as pl
from jax.experimental.pallas import tpu as pltpu


def _rollout_kernel(x0_ref, noise_ref,
                    w1a_ref, w2_ref, b2_ref, w3_ref, b3_ref,
                    resid_ref, nscr_ref, xst_ref,
                    *, sigma: float, F: int, C: int):
    BB = x0_ref.shape[0]
    M = BB // C
    H1 = w1a_ref.shape[1]
    H2 = w2_ref.shape[1]

    @pl.when(pl.program_id(0) == 0)
    def _init():
        pad = jnp.concatenate(
            [jnp.ones((BB, 1), jnp.float32),
             jnp.zeros((BB, F - 1), jnp.float32)], axis=1)
        xst_ref[...] = jnp.concatenate([x0_ref[...], pad], axis=1)

    w1a = w1a_ref[...]
    w2 = w2_ref[...]
    w3 = w3_ref[...]
    b2 = jnp.broadcast_to(b2_ref[...], (M, H2))

    for c in range(C):
        nscr_ref[c * M:(c + 1) * M, :] = (
            noise_ref[c * M:(c + 1) * M, :, :]
            .reshape(M, 16 * F)) * sigma + jnp.tile(b3_ref[...], (1, 16))

    xs = [xst_ref[c * M:(c + 1) * M, :] for c in range(C)]
    zpad = jnp.zeros((M, F), jnp.float32)

    preds = [[] for _ in range(C)]
    for s in range(16):
        h2s = []
        for c in range(C):
            h1 = jnp.maximum(
                jnp.dot(xs[c], w1a, preferred_element_type=jnp.float32),
                0.0)
            h2 = jnp.maximum(
                jnp.dot(h1, w2, preferred_element_type=jnp.float32)
                + b2, 0.0)
            h2s.append(h2)
        f_all = jnp.dot(jnp.concatenate(h2s, axis=0), w3,
                        preferred_element_type=jnp.float32)
        for c in range(C):
            f_pre = f_all[c * M:(c + 1) * M, :]
            preds[c].append(f_pre)
            upd = f_pre + nscr_ref[c * M:(c + 1) * M,
                                   s * F:(s + 1) * F]
            xs[c] = xs[c] + jnp.concatenate([upd, zpad], axis=1)

    for c in range(C):
        xst_ref[c * M:(c + 1) * M, :] = xs[c]
        resid_ref[c * M:(c + 1) * M, :, :] = (
            jnp.stack(preds[c], axis=1) + b3_ref[...][None, :, :])


def kernel(x, w1, b1, w2, b2, w3, b3, noises):
    sigma = 0.01
    B, L, F = x.shape
    T = L - 1
    H1 = w1.shape[1]
    H2 = w2.shape[1]

    x0 = x[:, 0, :]
    w1a = jnp.concatenate(
        [w1, b1, jnp.zeros((F - 1, H1), jnp.float32)], axis=0)

    _kern = functools.partial(_rollout_kernel, sigma=float(sigma), F=F, C=2)

    resid = pl.pallas_call(
        _kern,
        out_shape=jax.ShapeDtypeStruct((B, T, F), jnp.float32),
        grid=(T // 16,),
        in_specs=[
            pl.BlockSpec((B, F), lambda g: (0, 0)),
            pl.BlockSpec((B, 16, F), lambda g: (0, g, 0)),
            pl.BlockSpec((2 * F, H1), lambda g: (0, 0)),
            pl.BlockSpec((H1, H2), lambda g: (0, 0)),
            pl.BlockSpec((1, H2), lambda g: (0, 0)),
            pl.BlockSpec((H2, F), lambda g: (0, 0)),
            pl.BlockSpec((1, F), lambda g: (0, 0)),
        ],
        out_specs=pl.BlockSpec((B, 16, F), lambda g: (0, g, 0)),
        scratch_shapes=[pltpu.VMEM((B, 16 * F), jnp.float32),
                        pltpu.VMEM((B, 2 * F), jnp.float32)],
        compiler_params=pltpu.CompilerParams(
            dimension_semantics=("arbitrary",),
        ),
    )(x0, noises, w1a, w2, b2, w3, b3)

    return resid


# speedup vs baseline: 1.0306x; 1.0306x over previous
import functools

import jax
import jax.numpy as jnp
from jax.experimental import pallas as pl
from jax.experimental.pallas import tpu as pltpu


def _rollout_kernel(x0_ref, noise_ref,
                    w1a_ref, w2_ref, b2_ref, w3_ref, b3_ref,
                    resid_ref, nscr_ref, xst_ref,
                    *, sigma: float, F: int, C: int):
    BB = x0_ref.shape[0]
    M = BB // C
    H1 = w1a_ref.shape[1]
    H2 = w2_ref.shape[1]

    @pl.when(pl.program_id(0) == 0)
    def _init():
        pad = jnp.concatenate(
            [jnp.ones((BB, 1), jnp.float32),
             jnp.zeros((BB, F - 1), jnp.float32)], axis=1)
        xst_ref[...] = jnp.concatenate([x0_ref[...], pad], axis=1)

    w1a = w1a_ref[...]
    w2 = w2_ref[...]
    w3 = w3_ref[...]
    b2 = jnp.broadcast_to(b2_ref[...], (M, H2))

    for c in range(C):
        nscr_ref[c * M:(c + 1) * M, :] = (
            noise_ref[c * M:(c + 1) * M, :, :]
            .reshape(M, 8 * F)) * sigma + jnp.tile(b3_ref[...], (1, 8))

    xs = [xst_ref[c * M:(c + 1) * M, :] for c in range(C)]
    zpad = jnp.zeros((M, F), jnp.float32)

    preds = [[] for _ in range(C)]
    for s in range(8):
        h2s = []
        for c in range(C):
            h1 = jnp.maximum(
                jnp.dot(xs[c], w1a, preferred_element_type=jnp.float32),
                0.0)
            h2 = jnp.maximum(
                jnp.dot(h1, w2, preferred_element_type=jnp.float32)
                + b2, 0.0)
            h2s.append(h2)
        f_all = jnp.dot(jnp.concatenate(h2s, axis=0), w3,
                        preferred_element_type=jnp.float32)
        for c in range(C):
            f_pre = f_all[c * M:(c + 1) * M, :]
            preds[c].append(f_pre)
            upd = f_pre + nscr_ref[c * M:(c + 1) * M,
                                   s * F:(s + 1) * F]
            xs[c] = xs[c] + jnp.concatenate([upd, zpad], axis=1)

    for c in range(C):
        xst_ref[c * M:(c + 1) * M, :] = xs[c]
        resid_ref[c * M:(c + 1) * M, :, :] = (
            jnp.stack(preds[c], axis=1) + b3_ref[...][None, :, :])


def kernel(x, w1, b1, w2, b2, w3, b3, noises):
    sigma = 0.01
    B, L, F = x.shape
    T = L - 1
    H1 = w1.shape[1]
    H2 = w2.shape[1]

    x0 = x[:, 0, :]
    w1a = jnp.concatenate(
        [w1, b1, jnp.zeros((F - 1, H1), jnp.float32)], axis=0)

    _kern = functools.partial(_rollout_kernel, sigma=float(sigma), F=F, C=2)

    resid = pl.pallas_call(
        _kern,
        out_shape=jax.ShapeDtypeStruct((B, T, F), jnp.float32),
        grid=(T // 8,),
        in_specs=[
            pl.BlockSpec((B, F), lambda g: (0, 0)),
            pl.BlockSpec((B, 8, F), lambda g: (0, g, 0)),
            pl.BlockSpec((2 * F, H1), lambda g: (0, 0)),
            pl.BlockSpec((H1, H2), lambda g: (0, 0)),
            pl.BlockSpec((1, H2), lambda g: (0, 0)),
            pl.BlockSpec((H2, F), lambda g: (0, 0)),
            pl.BlockSpec((1, F), lambda g: (0, 0)),
        ],
        out_specs=pl.BlockSpec((B, 8, F), lambda g: (0, g, 0)),
        scratch_shapes=[pltpu.VMEM((B, 8 * F), jnp.float32),
                        pltpu.VMEM((B, 2 * F), jnp.float32)],
        compiler_params=pltpu.CompilerParams(
            dimension_semantics=("arbitrary",),
        ),
    )(x0, noises, w1a, w2, b2, w3, b3)

    return resid
